# baseline (speedup 1.0000x reference)
import jax
import jax.numpy as jnp
from jax.experimental import pallas as pl
from jax.experimental.pallas import tpu as pltpu

B = 32
SEQ = 512
PRED = 192
D = 32
H = 4
CD = 32
K = 512
COMMIT_W = 1.0
ORTHO_W = 0.8


def _fused_kernel(x_ref, w_in2_ref, b_in2_ref, w_out_ref, b_out_ref,
                  cb_ref, w_lin_ref, b_lin_ref,
                  out_ref, aux_ref, cnorm_ref):
    i = pl.program_id(0)
    commit = jnp.float32(0.0)

    @pl.when(i == 0)
    def _prep():
        for h in range(H):
            cbh = cb_ref[h]
            cnorm_ref[h, :] = jnp.sum(cbh * cbh, axis=1)

    for b2 in range(2):
        xb = x_ref[b2]
        last = xb[SEQ - 1:SEQ, :]
        x0 = xb - last
        xps = (jnp.dot(x0, w_in2_ref[...],
                       preferred_element_type=jnp.float32)
               + b_in2_ref[...])

        commit = commit + jnp.sum(xps * xps) * 0.25
        quants = []
        for h in range(H):
            xhs = xps[:, h * CD:(h + 1) * CD]
            cb = cb_ref[h]
            dots2 = jnp.dot(xhs, cb.T,
                            preferred_element_type=jnp.float32)
            dist2 = dots2 - cnorm_ref[h:h + 1, :]
            m = jnp.max(dist2, axis=1, keepdims=True)
            onehot = (dist2 == m).astype(jnp.float32)
            quant = jnp.dot(onehot, cb,
                            preferred_element_type=jnp.float32)
            commit = commit - jnp.sum(m)
            quants.append(quant)

        q = jnp.concatenate(quants, axis=1)
        qo = (jnp.dot(q, w_out_ref[...],
                      preferred_element_type=jnp.float32)
              + b_out_ref[...])
        y = jnp.dot(w_lin_ref[...], qo,
                    preferred_element_type=jnp.float32)
        out_ref[b2] = y + b_lin_ref[...] + last

    lane = jax.lax.broadcasted_iota(jnp.int32, (128,), 0)
    aux_ref[0, 0, :] = jnp.where(lane == 0, commit, 0.0)

    @pl.when(i < H)
    def _ortho():
        cb = cb_ref[i]
        norm = jnp.sqrt(jnp.sum(cb * cb, axis=1, keepdims=True))
        normed = cb / norm
        cos = jnp.dot(normed, normed.T,
                      preferred_element_type=jnp.float32)
        osum = jnp.sum(cos * cos)
        aux_ref[0, 0, :] = (jnp.where(lane == 0, commit, 0.0)
                            + jnp.where(lane == 1, osum, 0.0))


@jax.jit
def kernel(x, W_in, b_in, W_out, b_out, codebook, W_lin, b_lin):
    out, aux = pl.pallas_call(
        _fused_kernel,
        grid=(B // 2,),
        in_specs=[
            pl.BlockSpec((2, SEQ, D), lambda i: (i, 0, 0)),
            pl.BlockSpec((D, H * CD), lambda i: (0, 0)),
            pl.BlockSpec((1, H * CD), lambda i: (0, 0)),
            pl.BlockSpec((H * CD, D), lambda i: (0, 0)),
            pl.BlockSpec((1, D), lambda i: (0, 0)),
            pl.BlockSpec((H, K, CD), lambda i: (0, 0, 0)),
            pl.BlockSpec((PRED, SEQ), lambda i: (0, 0)),
            pl.BlockSpec((PRED, 1), lambda i: (0, 0)),
        ],
        out_specs=[
            pl.BlockSpec((2, PRED, D), lambda i: (i, 0, 0)),
            pl.BlockSpec((1, 1, 128), lambda i: (i, 0, 0)),
        ],
        out_shape=[
            jax.ShapeDtypeStruct((B, PRED, D), jnp.float32),
            jax.ShapeDtypeStruct((B // 2, 1, 128), jnp.float32),
        ],
        scratch_shapes=[pltpu.VMEM((H, K), jnp.float32)],
    )(x, W_in + W_in, (b_in + b_in).reshape(1, H * CD),
      W_out, b_out.reshape(1, D),
      codebook, W_lin, b_lin.reshape(PRED, 1))

    commit = jnp.sum(aux[:, 0, 0]) / (B * H * SEQ * CD)
    ortho = jnp.sum(aux[:H, 0, 1]) / (H * K * K) - 1.0 / K
    loss = COMMIT_W * commit + ORTHO_W * ortho
    return out, loss


# speedup vs baseline: 1.4680x; 1.4680x over previous
import jax
import jax.numpy as jnp
from jax.experimental import pallas as pl
from jax.experimental.pallas import tpu as pltpu

B = 32
SEQ = 512
PRED = 192
D = 32
H = 4
CD = 32
K = 512
COMMIT_W = 1.0
ORTHO_W = 0.8


def _fused_kernel(x_ref, w_in2_ref, b_in2_ref, w_out_ref, b_out_ref,
                  cb_ref, w_lin_ref, b_lin_ref,
                  out_ref, aux_ref, cnorm_ref):
    i = pl.program_id(0)

    @pl.when(i == 0)
    def _prep():
        for h in range(H):
            cbh = cb_ref[h]
            cnorm_ref[h, :] = jnp.sum(cbh * cbh, axis=1)

    xb = x_ref[0]
    last = xb[SEQ - 1:SEQ, :]
    x0 = xb - last
    w2 = w_in2_ref[...] + w_in2_ref[...]
    b2 = b_in2_ref[...] + b_in2_ref[...]
    xps = (jnp.dot(x0, w2, preferred_element_type=jnp.float32)
           + b2)

    commit = jnp.sum(xps * xps) * 0.25
    quants = []
    for h in range(H):
        xhs = xps[:, h * CD:(h + 1) * CD]
        cb = cb_ref[h]
        dots2 = jnp.dot(xhs, cb.T,
                        preferred_element_type=jnp.float32)
        dist2 = dots2 - cnorm_ref[h:h + 1, :]
        m = jnp.max(dist2, axis=1, keepdims=True)
        onehot = (dist2 == m).astype(jnp.float32)
        quant = jnp.dot(onehot, cb,
                        preferred_element_type=jnp.float32)
        commit = commit - jnp.sum(m)
        quants.append(quant)

    q = jnp.concatenate(quants, axis=1)
    qo = (jnp.dot(q, w_out_ref[...],
                  preferred_element_type=jnp.float32)
          + b_out_ref[...])
    y = jnp.dot(w_lin_ref[...], qo,
                preferred_element_type=jnp.float32)
    out_ref[0] = y + b_lin_ref[...] + last

    lane = jax.lax.broadcasted_iota(jnp.int32, (128,), 0)
    aux_ref[0, 0, :] = jnp.where(lane == 0, commit, 0.0)

    @pl.when(i < H)
    def _ortho():
        cb = cb_ref[i]
        norm = jnp.sqrt(jnp.sum(cb * cb, axis=1, keepdims=True))
        normed = cb / norm
        cos = jnp.dot(normed, normed.T,
                      preferred_element_type=jnp.float32)
        osum = jnp.sum(cos * cos)
        aux_ref[0, 0, :] = (jnp.where(lane == 0, commit, 0.0)
                            + jnp.where(lane == 1, osum, 0.0))


@jax.jit
def kernel(x, W_in, b_in, W_out, b_out, codebook, W_lin, b_lin):
    out, aux = pl.pallas_call(
        _fused_kernel,
        grid=(B,),
        in_specs=[
            pl.BlockSpec((1, SEQ, D), lambda i: (i, 0, 0)),
            pl.BlockSpec((D, H * CD), lambda i: (0, 0)),
            pl.BlockSpec((1, H * CD), lambda i: (0, 0)),
            pl.BlockSpec((H * CD, D), lambda i: (0, 0)),
            pl.BlockSpec((1, D), lambda i: (0, 0)),
            pl.BlockSpec((H, K, CD), lambda i: (0, 0, 0)),
            pl.BlockSpec((PRED, SEQ), lambda i: (0, 0)),
            pl.BlockSpec((PRED, 1), lambda i: (0, 0)),
        ],
        out_specs=[
            pl.BlockSpec((1, PRED, D), lambda i: (i, 0, 0)),
            pl.BlockSpec((1, 1, 128), lambda i: (i, 0, 0)),
        ],
        out_shape=[
            jax.ShapeDtypeStruct((B, PRED, D), jnp.float32),
            jax.ShapeDtypeStruct((B, 1, 128), jnp.float32),
        ],
        scratch_shapes=[pltpu.VMEM((H, K), jnp.float32)],
    )(x, W_in, b_in.reshape(1, H * CD),
      W_out, b_out.reshape(1, D),
      codebook, W_lin, b_lin.reshape(PRED, 1))

    commit = jnp.sum(aux[:, 0, 0]) / (B * H * SEQ * CD)
    ortho = jnp.sum(aux[:H, 0, 1]) / (H * K * K) - 1.0 / K
    loss = COMMIT_W * commit + ORTHO_W * ortho
    return out, loss


# speedup vs baseline: 1.5435x; 1.0514x over previous
import jax
import jax.numpy as jnp
from jax.experimental import pallas as pl
from jax.experimental.pallas import tpu as pltpu

B = 32
SEQ = 512
PRED = 192
D = 32
H = 4
CD = 32
K = 512
COMMIT_W = 1.0
ORTHO_W = 0.8


def _fused_kernel(x_ref, w_in2_ref, b_in2_ref, w_out_ref, b_out_ref,
                  cb_ref, w_lin_ref, b_lin_ref,
                  out_ref, aux_ref, cnorm_ref, acc_ref):
    i = pl.program_id(0)

    @pl.when(i == 0)
    def _prep():
        for h in range(H):
            cbh = cb_ref[h]
            cnorm_ref[h, :] = jnp.sum(cbh * cbh, axis=1)

    xb = x_ref[0]
    last = xb[SEQ - 1:SEQ, :]
    x0 = xb - last
    w2 = w_in2_ref[...] + w_in2_ref[...]
    b2 = b_in2_ref[...] + b_in2_ref[...]
    xps = (jnp.dot(x0, w2, preferred_element_type=jnp.float32)
           + b2)

    commit = jnp.sum(xps * xps) * 0.25
    quants = []
    for h in range(H):
        xhs = xps[:, h * CD:(h + 1) * CD]
        cb = cb_ref[h]
        dots2 = jnp.dot(xhs, cb.T,
                        preferred_element_type=jnp.float32)
        dist2 = dots2 - cnorm_ref[h:h + 1, :]
        m = jnp.max(dist2, axis=1, keepdims=True)
        onehot = (dist2 == m).astype(jnp.float32)
        quant = jnp.dot(onehot, cb,
                        preferred_element_type=jnp.float32)
        commit = commit - jnp.sum(m)
        quants.append(quant)

    q = jnp.concatenate(quants, axis=1)
    qo = (jnp.dot(q, w_out_ref[...],
                  preferred_element_type=jnp.float32)
          + b_out_ref[...])
    y = jnp.dot(w_lin_ref[...], qo,
                preferred_element_type=jnp.float32)
    out_ref[0] = y + b_lin_ref[...] + last

    lane = jax.lax.broadcasted_iota(jnp.int32, (128,), 0)
    row = jnp.where(lane == 0, commit, 0.0)

    @pl.when(i < H)
    def _ortho():
        cb = cb_ref[i]
        norm = jnp.sqrt(jnp.sum(cb * cb, axis=1, keepdims=True))
        normed = cb / norm
        cos = jnp.dot(normed, normed.T,
                      preferred_element_type=jnp.float32)
        osum = jnp.sum(cos * cos)
        acc_ref[0, :] = (jnp.where(i == 0, 0.0, acc_ref[0, :])
                         + row + jnp.where(lane == 1, osum, 0.0))

    @pl.when(jnp.logical_and(i >= H, i < B - 1))
    def _acc():
        acc_ref[0, :] = acc_ref[0, :] + row

    @pl.when(i == B - 1)
    def _final():
        acc = acc_ref[0, :] + row
        commit_total = jnp.sum(jnp.where(lane == 0, acc, 0.0))
        ortho_total = jnp.sum(jnp.where(lane == 1, acc, 0.0))
        loss = (COMMIT_W * commit_total / (B * H * SEQ * CD)
                + ORTHO_W * (ortho_total / (H * K * K) - 1.0 / K))
        aux_ref[0, 0, :] = jnp.where(lane == 0, loss, 0.0)


@jax.jit
def kernel(x, W_in, b_in, W_out, b_out, codebook, W_lin, b_lin):
    out, aux = pl.pallas_call(
        _fused_kernel,
        grid=(B,),
        in_specs=[
            pl.BlockSpec((1, SEQ, D), lambda i: (i, 0, 0)),
            pl.BlockSpec((D, H * CD), lambda i: (0, 0)),
            pl.BlockSpec((1, H * CD), lambda i: (0, 0)),
            pl.BlockSpec((H * CD, D), lambda i: (0, 0)),
            pl.BlockSpec((1, D), lambda i: (0, 0)),
            pl.BlockSpec((H, K, CD), lambda i: (0, 0, 0)),
            pl.BlockSpec((PRED, SEQ), lambda i: (0, 0)),
            pl.BlockSpec((PRED, 1), lambda i: (0, 0)),
        ],
        out_specs=[
            pl.BlockSpec((1, PRED, D), lambda i: (i, 0, 0)),
            pl.BlockSpec((1, 1, 128), lambda i: (0, 0, 0)),
        ],
        out_shape=[
            jax.ShapeDtypeStruct((B, PRED, D), jnp.float32),
            jax.ShapeDtypeStruct((1, 1, 128), jnp.float32),
        ],
        scratch_shapes=[pltpu.VMEM((H, K), jnp.float32),
                        pltpu.VMEM((1, 128), jnp.float32)],
    )(x, W_in, b_in.reshape(1, H * CD),
      W_out, b_out.reshape(1, D),
      codebook, W_lin, b_lin.reshape(PRED, 1))

    return out, aux[0, 0, 0]


# speedup vs baseline: 1.5441x; 1.0004x over previous
import jax
import jax.numpy as jnp
from jax.experimental import pallas as pl
from jax.experimental.pallas import tpu as pltpu

B = 32
SEQ = 512
PRED = 192
D = 32
H = 4
CD = 32
K = 512
COMMIT_W = 1.0
ORTHO_W = 0.8


def _fused_kernel(x_ref, w_in_ref, b_in_ref, w_out_ref, b_out_ref,
                  cb_ref, w_lin_ref, b_lin_ref,
                  out_ref, aux_ref, cnorm_ref, acc_ref):
    i = pl.program_id(0)

    @pl.when(i == 0)
    def _prep():
        for h in range(H):
            cbh = cb_ref[h]
            cnorm_ref[h, :] = jnp.sum(cbh * cbh, axis=1)

    xb = x_ref[0]
    last = xb[SEQ - 1:SEQ, :]
    x0 = xb - last
    w2 = w_in_ref[...] + w_in_ref[...]
    b2 = b_in_ref[...] + b_in_ref[...]
    xps = (jnp.dot(x0, w2, preferred_element_type=jnp.float32)
           + b2)

    commit = jnp.sum(xps * xps) * 0.25
    quants = []
    for h in range(H):
        xhs = xps[:, h * CD:(h + 1) * CD]
        cb = cb_ref[h]
        dots2 = jnp.dot(xhs, cb.T,
                        preferred_element_type=jnp.float32)
        dist2 = dots2 - cnorm_ref[h:h + 1, :]
        m = jnp.max(dist2, axis=1, keepdims=True)
        onehot = (dist2 == m).astype(jnp.float32)
        quant = jnp.dot(onehot, cb,
                        preferred_element_type=jnp.float32)
        commit = commit - jnp.sum(m)
        quants.append(quant)

    q = jnp.concatenate(quants, axis=1)
    qo = (jnp.dot(q, w_out_ref[...],
                  preferred_element_type=jnp.float32)
          + b_out_ref[...])
    y = jnp.dot(w_lin_ref[...], qo,
                preferred_element_type=jnp.float32)
    out_ref[0] = y + b_lin_ref[...] + last

    lane = jax.lax.broadcasted_iota(jnp.int32, (128,), 0)
    row = jnp.where(lane == 0, commit, 0.0)

    @pl.when(i < H)
    def _ortho():
        cb = cb_ref[i]
        norm = jnp.sqrt(jnp.sum(cb * cb, axis=1, keepdims=True))
        normed = cb / norm
        cos = jnp.dot(normed, normed.T,
                      preferred_element_type=jnp.float32)
        osum = jnp.sum(cos * cos)
        acc_ref[0, :] = (jnp.where(i == 0, 0.0, acc_ref[0, :])
                         + row + jnp.where(lane == 1, osum, 0.0))

    @pl.when(jnp.logical_and(i >= H, i < B - 1))
    def _acc():
        acc_ref[0, :] = acc_ref[0, :] + row

    @pl.when(i == B - 1)
    def _final():
        acc = acc_ref[0, :] + row
        commit_total = jnp.sum(jnp.where(lane == 0, acc, 0.0))
        ortho_total = jnp.sum(jnp.where(lane == 1, acc, 0.0))
        loss = (COMMIT_W * commit_total / (B * H * SEQ * CD)
                + ORTHO_W * (ortho_total / (H * K * K) - 1.0 / K))
        aux_ref[0, 0, :] = jnp.where(lane == 0, loss, 0.0)


@jax.jit
def kernel(x, W_in, b_in, W_out, b_out, codebook, W_lin, b_lin):
    out, aux = pl.pallas_call(
        _fused_kernel,
        grid=(B,),
        in_specs=[
            pl.BlockSpec((1, SEQ, D), lambda i: (i, 0, 0)),
            pl.BlockSpec((D, H * CD), lambda i: (0, 0)),
            pl.BlockSpec((1, H * CD), lambda i: (0, 0)),
            pl.BlockSpec((H * CD, D), lambda i: (0, 0)),
            pl.BlockSpec((1, D), lambda i: (0, 0)),
            pl.BlockSpec((H, K, CD), lambda i: (0, 0, 0)),
            pl.BlockSpec((PRED, SEQ), lambda i: (0, 0)),
            pl.BlockSpec((PRED, 1), lambda i: (0, 0)),
        ],
        out_specs=[
            pl.BlockSpec((1, PRED, D), lambda i: (i, 0, 0)),
            pl.BlockSpec((1, 1, 128), lambda i: (0, 0, 0)),
        ],
        out_shape=[
            jax.ShapeDtypeStruct((B, PRED, D), jnp.float32),
            jax.ShapeDtypeStruct((1, 1, 128), jnp.float32),
        ],
        scratch_shapes=[pltpu.VMEM((H, K), jnp.float32),
                        pltpu.VMEM((1, 128), jnp.float32)],
    )(x, W_in, b_in.reshape(1, H * CD),
      W_out, b_out.reshape(1, D),
      codebook, W_lin, b_lin.reshape(PRED, 1))

    return out, aux[0, 0, 0]
